# baseline (speedup 1.0000x reference)
import functools

import jax
import jax.numpy as jnp
from jax.experimental import pallas as pl
from jax.experimental.pallas import tpu as pltpu


def _lane_shift(x, d, s):
    if d == 0:
        return x
    rows = x.shape[0]
    if d > 0:
        return jnp.concatenate(
            [x[:, d:], jnp.zeros((rows, d), x.dtype)], axis=1)
    return jnp.concatenate(
        [jnp.zeros((rows, -d), x.dtype), x[:, :s + d]], axis=1)


def _tap_matrix(x, mask_w0, mask_wL, W, S, sign):
    xm = {0: jnp.where(mask_wL if sign > 0 else mask_w0, 0.0, x),
          1: x,
          2: jnp.where(mask_w0 if sign > 0 else mask_wL, 0.0, x)}
    taps = []
    for kh in range(3):
        for kw in range(3):
            d = sign * ((kh - 1) * W + (kw - 1))
            taps.append(_lane_shift(xm[kw], d, S))
    return jnp.concatenate(taps, axis=0)


def _lrp_kernel(a_ref, r_ref, wm_ref, bn_ref,
                out_ref, et_ref, *, H, W, width, cout, cin, eps):
    S = H * W
    P, Q = H // 2, W // 2
    A = a_ref[0]

    rows = [2 * width + 2 * cout, 2 * width, width, 2 * cout, width, cin]
    offs = [sum(rows[:i]) for i in range(len(rows))]
    wfs_ref = wm_ref.at[offs[0]:offs[0] + rows[0], 0:cin]
    w2s_ref = wm_ref.at[offs[1]:offs[1] + rows[1], 0:9 * width]
    w2bk_ref = wm_ref.at[offs[2]:offs[2] + rows[2], 0:9 * width]
    w3s_ref = wm_ref.at[offs[3]:offs[3] + rows[3], 0:width]
    w3bp_ref = wm_ref.at[offs[4]:offs[4] + rows[4], 0:cout]
    wc_ref = wm_ref.at[offs[5]:offs[5] + rows[5], 0:width + cout]
    bo = [0, width, 2 * width, 3 * width, 4 * width, 4 * width + cout,
          4 * width + 2 * cout, 4 * width + 3 * cout]
    s1_ref = bn_ref.at[bo[0]:bo[0] + width, :]
    b1_ref = bn_ref.at[bo[1]:bo[1] + width, :]
    s2_ref = bn_ref.at[bo[2]:bo[2] + width, :]
    b2_ref = bn_ref.at[bo[3]:bo[3] + width, :]
    s3_ref = bn_ref.at[bo[4]:bo[4] + cout, :]
    b3_ref = bn_ref.at[bo[5]:bo[5] + cout, :]
    sd_ref = bn_ref.at[bo[6]:bo[6] + cout, :]
    bd_ref = bn_ref.at[bo[7]:bo[7] + cout, :]

    def mm(w, x):
        return jax.lax.dot_general(
            w, x, (((1,), (0,)), ((), ())),
            preferred_element_type=jnp.float32)

    @pl.when(pl.program_id(0) == 0)
    def _():
        si = jax.lax.broadcasted_iota(jnp.int32, (P * Q, S), 0)
        li = jax.lax.broadcasted_iota(jnp.int32, (P * Q, S), 1)
        tgt = (si // Q) * (2 * W) + (si % Q) * 2
        et_ref[...] = (li == tgt).astype(jnp.float32)

    Remb = mm(r_ref[0], et_ref[...])

    fs = mm(wfs_ref[...], A)
    h1 = jnp.maximum(fs[:width] * s1_ref[...] + b1_ref[...], 0.0)
    z1 = fs[width:2 * width]
    short = fs[2 * width:2 * width + cout] * sd_ref[...] + bd_ref[...]
    zd = fs[2 * width + cout:]

    wmod = jax.lax.broadcasted_iota(jnp.int32, (1, S), 1) % W
    mask_w0 = wmod == 0
    mask_wL = wmod == W - 1

    taps = _tap_matrix(h1, mask_w0, mask_wL, W, S, 1)
    z2 = mm(w2s_ref[...], taps)
    h2 = jnp.maximum(z2[:width] * s2_ref[...] + b2_ref[...], 0.0)
    z2p = z2[width:]

    mz3 = mm(w3s_ref[...], h2)
    mstream = mz3[:cout] * s3_ref[...] + b3_ref[...]
    z3 = mz3[cout:]
    am = jnp.abs(mstream)
    ratio = am / (am + jnp.abs(short) + 1e-30)
    r_main = ratio * Remb
    r_short = Remb - r_main

    s3v = r_main / (z3 + eps)
    r2 = h2 * mm(w3bp_ref[...], s3v)

    s2v = r2 / (z2p + eps)
    btaps = _tap_matrix(s2v, mask_w0, mask_wL, W, S, -1)
    c2 = mm(w2bk_ref[...], btaps)
    r1 = h1 * c2

    s1v = r1 / (z1 + eps)
    sdv = r_short / (zd + eps)
    sv = jnp.concatenate([s1v, sdv], axis=0)
    out_ref[0] = (A * mm(wc_ref[...], sv)).reshape(out_ref.shape[1:])


def kernel(a, r, w1, w2, w3, wd,
           bn1_g, bn1_b, bn1_mu, bn1_v,
           bn2_g, bn2_b, bn2_mu, bn2_v,
           bn3_g, bn3_b, bn3_mu, bn3_v,
           bnd_g, bnd_b, bnd_mu, bnd_v):
    eps = 1e-5
    bn_eps = 1e-5
    n, cin, h, w = a.shape
    cout = r.shape[1]
    width = w1.shape[0]
    S = h * w

    A = a.reshape(n, cin, S)
    rf = r.reshape(n, cout, (h // 2) * (w // 2))

    w1m = w1.reshape(width, cin)
    w1mp = jnp.maximum(w1m, 0.0)
    wdm = wd.reshape(cout, cin)
    wdmp = jnp.maximum(wdm, 0.0)
    wfs = jnp.concatenate([w1m, w1mp, wdm, wdmp], axis=0)
    w3m = w3.reshape(cout, width)
    w3mp = jnp.maximum(w3m, 0.0)
    w3s = jnp.concatenate([w3m, w3mp], axis=0)
    w3bp = w3mp.T
    w2k = w2.transpose(0, 2, 3, 1).reshape(width, 9 * width)
    w2s = jnp.concatenate([w2k, jnp.maximum(w2k, 0.0)], axis=0)
    w2bk = jnp.maximum(w2, 0.0).transpose(1, 2, 3, 0).reshape(width, 9 * width)
    wc = jnp.concatenate([w1mp.T, wdmp.T], axis=1)

    def bn_fold(g, b, mu, v):
        s = g / jnp.sqrt(v + bn_eps)
        return s, b - mu * s

    s1, b1 = bn_fold(bn1_g, bn1_b, bn1_mu, bn1_v)
    s2, b2 = bn_fold(bn2_g, bn2_b, bn2_mu, bn2_v)
    s3, b3 = bn_fold(bn3_g, bn3_b, bn3_mu, bn3_v)
    sd, bd = bn_fold(bnd_g, bnd_b, bnd_mu, bnd_v)

    wcols = max(cin, 9 * width, width + cout)
    mats = [wfs, w2s, w2bk, w3s, w3bp, wc]
    wm = jnp.concatenate(
        [jnp.pad(m, ((0, 0), (0, wcols - m.shape[1]))) for m in mats],
        axis=0)
    bn = jnp.concatenate([s1, b1, s2, b2, s3, b3, sd, bd]).reshape(-1, 1)

    def fixed(x):
        return pl.BlockSpec(x.shape, lambda i: (0,) * x.ndim)

    weights = [wm, bn]

    out = pl.pallas_call(
        functools.partial(_lrp_kernel, H=h, W=w, width=width, cout=cout,
                          cin=cin, eps=eps),
        out_shape=jax.ShapeDtypeStruct((n, cin, h, w), jnp.float32),
        grid=(n,),
        in_specs=[pl.BlockSpec((1, cin, S), lambda i: (i, 0, 0)),
                  pl.BlockSpec((1, cout, (h // 2) * (w // 2)),
                               lambda i: (i, 0, 0))]
                 + [fixed(x) for x in weights],
        out_specs=pl.BlockSpec((1, cin, h, w), lambda i: (i, 0, 0, 0)),
        scratch_shapes=[pltpu.VMEM(((h // 2) * (w // 2), S), jnp.float32)],
        compiler_params=pltpu.CompilerParams(
            dimension_semantics=("arbitrary",),
            vmem_limit_bytes=100 * 1024 * 1024),
    )(A, rf, *weights)

    return out


# speedup vs baseline: 1.1922x; 1.1922x over previous
import functools

import jax
import jax.numpy as jnp
from jax.experimental import pallas as pl
from jax.experimental.pallas import tpu as pltpu


def _lane_shift(x, d, s):
    if d == 0:
        return x
    rows = x.shape[0]
    if d > 0:
        return jnp.concatenate(
            [x[:, d:], jnp.zeros((rows, d), x.dtype)], axis=1)
    return jnp.concatenate(
        [jnp.zeros((rows, -d), x.dtype), x[:, :s + d]], axis=1)


def _tap_matrix(x, mask_w0, mask_wL, W, S, sign):
    xm = {0: jnp.where(mask_wL if sign > 0 else mask_w0, 0.0, x),
          1: x,
          2: jnp.where(mask_w0 if sign > 0 else mask_wL, 0.0, x)}
    taps = []
    for kh in range(3):
        for kw in range(3):
            d = sign * ((kh - 1) * W + (kw - 1))
            taps.append(_lane_shift(xm[kw], d, S))
    return jnp.concatenate(taps, axis=0)


def _lrp_kernel(a_ref, r_ref, wm_ref, bn_ref,
                out_ref, et_ref, *, H, W, width, cout, cin, eps):
    S = H * W
    P, Q = H // 2, W // 2
    A = a_ref[0]

    rows = [2 * width + 2 * cout, 2 * width, width, 2 * cout, width, cin]
    offs = [sum(rows[:i]) for i in range(len(rows))]
    wfs_ref = wm_ref.at[offs[0]:offs[0] + rows[0], 0:cin]
    w2s_ref = wm_ref.at[offs[1]:offs[1] + rows[1], 0:9 * width]
    w2bk_ref = wm_ref.at[offs[2]:offs[2] + rows[2], 0:9 * width]
    w3s_ref = wm_ref.at[offs[3]:offs[3] + rows[3], 0:width]
    w3bp_ref = wm_ref.at[offs[4]:offs[4] + rows[4], 0:cout]
    wc_ref = wm_ref.at[offs[5]:offs[5] + rows[5], 0:width + cout]
    bo = [0, width, 2 * width, 3 * width, 4 * width, 4 * width + cout,
          4 * width + 2 * cout, 4 * width + 3 * cout]
    s1_ref = bn_ref.at[bo[0]:bo[0] + width, :]
    b1_ref = bn_ref.at[bo[1]:bo[1] + width, :]
    s2_ref = bn_ref.at[bo[2]:bo[2] + width, :]
    b2_ref = bn_ref.at[bo[3]:bo[3] + width, :]
    s3_ref = bn_ref.at[bo[4]:bo[4] + cout, :]
    b3_ref = bn_ref.at[bo[5]:bo[5] + cout, :]
    sd_ref = bn_ref.at[bo[6]:bo[6] + cout, :]
    bd_ref = bn_ref.at[bo[7]:bo[7] + cout, :]

    def mm(w, x):
        return jax.lax.dot_general(
            w, x, (((1,), (0,)), ((), ())),
            preferred_element_type=jnp.float32)

    @pl.when(pl.program_id(0) == 0)
    def _():
        si = jax.lax.broadcasted_iota(jnp.int32, (P * Q, S), 0)
        li = jax.lax.broadcasted_iota(jnp.int32, (P * Q, S), 1)
        tgt = (si // Q) * (2 * W) + (si % Q) * 2
        et_ref[...] = (li == tgt).astype(jnp.float32)

    Remb = mm(r_ref[0], et_ref[...])

    fs = mm(wfs_ref[...], A)
    h1 = jnp.maximum(fs[:width] * s1_ref[...] + b1_ref[...], 0.0)
    z1 = fs[width:2 * width]
    short = fs[2 * width:2 * width + cout] * sd_ref[...] + bd_ref[...]
    zd = fs[2 * width + cout:]

    wmod = jax.lax.broadcasted_iota(jnp.int32, (1, S), 1) % W
    mask_w0 = wmod == 0
    mask_wL = wmod == W - 1

    taps = _tap_matrix(h1, mask_w0, mask_wL, W, S, 1)
    z2 = mm(w2s_ref[...], taps)
    h2 = jnp.maximum(z2[:width] * s2_ref[...] + b2_ref[...], 0.0)
    z2p = z2[width:]

    mz3 = mm(w3s_ref[...], h2)
    mstream = mz3[:cout] * s3_ref[...] + b3_ref[...]
    z3 = mz3[cout:]
    am = jnp.abs(mstream)
    ratio = am / (am + jnp.abs(short) + 1e-30)
    r_main = ratio * Remb
    r_short = Remb - r_main

    s3v = r_main / (z3 + eps)
    r2 = h2 * mm(w3bp_ref[...], s3v)

    s2v = r2 / (z2p + eps)
    btaps = _tap_matrix(s2v, mask_w0, mask_wL, W, S, -1)
    c2 = mm(w2bk_ref[...], btaps)
    r1 = h1 * c2

    s1v = r1 / (z1 + eps)
    sdv = r_short / (zd + eps)
    sv = jnp.concatenate([s1v, sdv], axis=0)
    out_ref[0] = A * mm(wc_ref[...], sv)


def kernel(a, r, w1, w2, w3, wd,
           bn1_g, bn1_b, bn1_mu, bn1_v,
           bn2_g, bn2_b, bn2_mu, bn2_v,
           bn3_g, bn3_b, bn3_mu, bn3_v,
           bnd_g, bnd_b, bnd_mu, bnd_v):
    eps = 1e-5
    bn_eps = 1e-5
    n, cin, h, w = a.shape
    cout = r.shape[1]
    width = w1.shape[0]
    S = h * w

    A = a.reshape(n, cin, S)
    rf = r.reshape(n, cout, (h // 2) * (w // 2))

    w1m = w1.reshape(width, cin)
    w1mp = jnp.maximum(w1m, 0.0)
    wdm = wd.reshape(cout, cin)
    wdmp = jnp.maximum(wdm, 0.0)
    wfs = jnp.concatenate([w1m, w1mp, wdm, wdmp], axis=0)
    w3m = w3.reshape(cout, width)
    w3mp = jnp.maximum(w3m, 0.0)
    w3s = jnp.concatenate([w3m, w3mp], axis=0)
    w3bp = w3mp.T
    w2k = w2.transpose(0, 2, 3, 1).reshape(width, 9 * width)
    w2s = jnp.concatenate([w2k, jnp.maximum(w2k, 0.0)], axis=0)
    w2bk = jnp.maximum(w2, 0.0).transpose(1, 2, 3, 0).reshape(width, 9 * width)
    wc = jnp.concatenate([w1mp.T, wdmp.T], axis=1)

    def bn_fold(g, b, mu, v):
        s = g / jnp.sqrt(v + bn_eps)
        return s, b - mu * s

    s1, b1 = bn_fold(bn1_g, bn1_b, bn1_mu, bn1_v)
    s2, b2 = bn_fold(bn2_g, bn2_b, bn2_mu, bn2_v)
    s3, b3 = bn_fold(bn3_g, bn3_b, bn3_mu, bn3_v)
    sd, bd = bn_fold(bnd_g, bnd_b, bnd_mu, bnd_v)

    wcols = max(cin, 9 * width, width + cout)
    mats = [wfs, w2s, w2bk, w3s, w3bp, wc]
    wm = jnp.concatenate(
        [jnp.pad(m, ((0, 0), (0, wcols - m.shape[1]))) for m in mats],
        axis=0)
    bn = jnp.concatenate([s1, b1, s2, b2, s3, b3, sd, bd]).reshape(-1, 1)

    def fixed(x):
        return pl.BlockSpec(x.shape, lambda i: (0,) * x.ndim)

    weights = [wm, bn]

    out = pl.pallas_call(
        functools.partial(_lrp_kernel, H=h, W=w, width=width, cout=cout,
                          cin=cin, eps=eps),
        out_shape=jax.ShapeDtypeStruct((n, cin, S), jnp.float32),
        grid=(n,),
        in_specs=[pl.BlockSpec((1, cin, S), lambda i: (i, 0, 0)),
                  pl.BlockSpec((1, cout, (h // 2) * (w // 2)),
                               lambda i: (i, 0, 0))]
                 + [fixed(x) for x in weights],
        out_specs=pl.BlockSpec((1, cin, S), lambda i: (i, 0, 0)),
        scratch_shapes=[pltpu.VMEM(((h // 2) * (w // 2), S), jnp.float32)],
        compiler_params=pltpu.CompilerParams(
            dimension_semantics=("arbitrary",),
            vmem_limit_bytes=56 * 1024 * 1024),
    )(A, rf, *weights)

    return out.reshape(n, cin, h, w)


# speedup vs baseline: 1.1986x; 1.0054x over previous
import functools

import jax
import jax.numpy as jnp
from jax.experimental import pallas as pl
from jax.experimental.pallas import tpu as pltpu


def _lane_shift(x, d, s):
    if d == 0:
        return x
    rows = x.shape[0]
    if d > 0:
        return jnp.concatenate(
            [x[:, d:], jnp.zeros((rows, d), x.dtype)], axis=1)
    return jnp.concatenate(
        [jnp.zeros((rows, -d), x.dtype), x[:, :s + d]], axis=1)


def _tap_matrix(x, mask_w0, mask_wL, W, S, sign):
    xm = {0: jnp.where(mask_wL if sign > 0 else mask_w0, 0.0, x),
          1: x,
          2: jnp.where(mask_w0 if sign > 0 else mask_wL, 0.0, x)}
    taps = []
    for kh in range(3):
        for kw in range(3):
            d = sign * ((kh - 1) * W + (kw - 1))
            taps.append(_lane_shift(xm[kw], d, S))
    return jnp.concatenate(taps, axis=0)


def _lrp_kernel(a_ref, r_ref, wm_ref, bn_ref,
                out_ref, et_ref, *, H, W, width, cout, cin, eps):
    S = H * W
    P, Q = H // 2, W // 2
    A = a_ref[0]

    rows = [2 * width + 2 * cout, 2 * width, width, 2 * cout, width, cin]
    offs = [sum(rows[:i]) for i in range(len(rows))]
    wfs_ref = wm_ref.at[offs[0]:offs[0] + rows[0], 0:cin]
    w2s_ref = wm_ref.at[offs[1]:offs[1] + rows[1], 0:9 * width]
    w2bk_ref = wm_ref.at[offs[2]:offs[2] + rows[2], 0:9 * width]
    w3s_ref = wm_ref.at[offs[3]:offs[3] + rows[3], 0:width]
    w3bp_ref = wm_ref.at[offs[4]:offs[4] + rows[4], 0:cout]
    wc_ref = wm_ref.at[offs[5]:offs[5] + rows[5], 0:width + cout]
    bo = [0, width, 2 * width, 3 * width, 4 * width, 4 * width + cout,
          4 * width + 2 * cout, 4 * width + 3 * cout]
    s1_ref = bn_ref.at[bo[0]:bo[0] + width, :]
    b1_ref = bn_ref.at[bo[1]:bo[1] + width, :]
    s2_ref = bn_ref.at[bo[2]:bo[2] + width, :]
    b2_ref = bn_ref.at[bo[3]:bo[3] + width, :]
    s3_ref = bn_ref.at[bo[4]:bo[4] + cout, :]
    b3_ref = bn_ref.at[bo[5]:bo[5] + cout, :]
    sd_ref = bn_ref.at[bo[6]:bo[6] + cout, :]
    bd_ref = bn_ref.at[bo[7]:bo[7] + cout, :]

    def mm(w, x):
        return jax.lax.dot_general(
            w, x, (((1,), (0,)), ((), ())),
            preferred_element_type=jnp.float32)

    @pl.when(pl.program_id(0) == 0)
    def _():
        si = jax.lax.broadcasted_iota(jnp.int32, (P * Q, S), 0)
        li = jax.lax.broadcasted_iota(jnp.int32, (P * Q, S), 1)
        tgt = (si // Q) * (2 * W) + (si % Q) * 2
        et_ref[...] = (li == tgt).astype(jnp.float32)

    Remb = mm(r_ref[0], et_ref[...])

    fs = mm(wfs_ref[...], A)
    h1 = jnp.maximum(fs[:width] * s1_ref[...] + b1_ref[...], 0.0)
    z1 = fs[width:2 * width]
    short = fs[2 * width:2 * width + cout] * sd_ref[...] + bd_ref[...]
    zd = fs[2 * width + cout:]

    wmod = jax.lax.broadcasted_iota(jnp.int32, (1, S), 1) % W
    mask_w0 = wmod == 0
    mask_wL = wmod == W - 1

    taps = _tap_matrix(h1, mask_w0, mask_wL, W, S, 1)
    z2 = mm(w2s_ref[...], taps)
    h2 = jnp.maximum(z2[:width] * s2_ref[...] + b2_ref[...], 0.0)
    z2p = z2[width:]

    mz3 = mm(w3s_ref[...], h2)
    mstream = mz3[:cout] * s3_ref[...] + b3_ref[...]
    z3 = mz3[cout:]
    am = jnp.abs(mstream)
    ratio = am / (am + jnp.abs(short) + 1e-30)
    r_main = ratio * Remb
    r_short = Remb - r_main

    s3v = r_main / (z3 + eps)
    r2 = h2 * mm(w3bp_ref[...], s3v)

    s2v = r2 / (z2p + eps)
    btaps = _tap_matrix(s2v, mask_w0, mask_wL, W, S, -1)
    c2 = mm(w2bk_ref[...], btaps)
    r1 = h1 * c2

    s1v = r1 / (z1 + eps)
    sdv = r_short / (zd + eps)
    sv = jnp.concatenate([s1v, sdv], axis=0)
    out_ref[0] = A * mm(wc_ref[...], sv)


def kernel(a, r, w1, w2, w3, wd,
           bn1_g, bn1_b, bn1_mu, bn1_v,
           bn2_g, bn2_b, bn2_mu, bn2_v,
           bn3_g, bn3_b, bn3_mu, bn3_v,
           bnd_g, bnd_b, bnd_mu, bnd_v):
    eps = 1e-5
    bn_eps = 1e-5
    n, cin, h, w = a.shape
    cout = r.shape[1]
    width = w1.shape[0]
    S = h * w

    A = a.reshape(n, cin, S)
    rf = r.reshape(n, cout, (h // 2) * (w // 2))

    w1m = w1.reshape(width, cin)
    w1mp = jnp.maximum(w1m, 0.0)
    wdm = wd.reshape(cout, cin)
    wdmp = jnp.maximum(wdm, 0.0)
    wfs = jnp.concatenate([w1m, w1mp, wdm, wdmp], axis=0)
    w3m = w3.reshape(cout, width)
    w3mp = jnp.maximum(w3m, 0.0)
    w3s = jnp.concatenate([w3m, w3mp], axis=0)
    w3bp = w3mp.T
    w2k = w2.transpose(0, 2, 3, 1).reshape(width, 9 * width)
    w2s = jnp.concatenate([w2k, jnp.maximum(w2k, 0.0)], axis=0)
    w2bk = jnp.maximum(w2, 0.0).transpose(1, 2, 3, 0).reshape(width, 9 * width)
    wc = jnp.concatenate([w1mp.T, wdmp.T], axis=1)

    def bn_fold(g, b, mu, v):
        s = g / jnp.sqrt(v + bn_eps)
        return s, b - mu * s

    s1, b1 = bn_fold(bn1_g, bn1_b, bn1_mu, bn1_v)
    s2, b2 = bn_fold(bn2_g, bn2_b, bn2_mu, bn2_v)
    s3, b3 = bn_fold(bn3_g, bn3_b, bn3_mu, bn3_v)
    sd, bd = bn_fold(bnd_g, bnd_b, bnd_mu, bnd_v)

    wcols = max(cin, 9 * width, width + cout)
    mats = [wfs, w2s, w2bk, w3s, w3bp, wc]
    wm = jnp.concatenate(
        [jnp.pad(m, ((0, 0), (0, wcols - m.shape[1]))) for m in mats],
        axis=0)
    bn = jnp.concatenate([s1, b1, s2, b2, s3, b3, sd, bd]).reshape(-1, 1)

    def fixed(x):
        return pl.BlockSpec(x.shape, lambda i: (0,) * x.ndim)

    weights = [wm, bn]

    out = pl.pallas_call(
        functools.partial(_lrp_kernel, H=h, W=w, width=width, cout=cout,
                          cin=cin, eps=eps),
        out_shape=jax.ShapeDtypeStruct((n, cin, S), jnp.float32),
        grid=(n,),
        in_specs=[pl.BlockSpec((1, cin, S), lambda i: (i, 0, 0)),
                  pl.BlockSpec((1, cout, (h // 2) * (w // 2)),
                               lambda i: (i, 0, 0))]
                 + [fixed(x) for x in weights],
        out_specs=pl.BlockSpec((1, cin, S), lambda i: (i, 0, 0)),
        scratch_shapes=[pltpu.VMEM(((h // 2) * (w // 2), S), jnp.float32)],
        compiler_params=pltpu.CompilerParams(
            dimension_semantics=("arbitrary",),
            vmem_limit_bytes=100 * 1024 * 1024),
    )(A, rf, *weights)

    return out.reshape(n, cin, h, w)
